# baseline (speedup 1.0000x reference)
import functools

import jax
import jax.numpy as jnp
from jax import lax
from jax.experimental import pallas as pl
from jax.experimental.pallas import tpu as pltpu
from jax.experimental.pallas import tpu_sc as plsc

_BCAST_DNUMS = lax.GatherDimensionNumbers(
    offset_dims=(), collapsed_slice_dims=(0,), start_index_map=(0,))


def _bcast_lane(vec16, k):
    idx = jnp.full((16, 1), k, jnp.int32)
    return lax.gather(vec16, idx, _BCAST_DNUMS, (1,),
                      mode=lax.GatherScatterMode.PROMISE_IN_BOUNDS)


_G = 2048
_F = 128
_V = 100000
_CH = 80
_RPT = 50
_NT = 25
_NROW = _V // _CH


def _u_body(x_ref, g_ref, w1_ref, w2_ref, bl_ref, u_ref, c_ref):
    b = pl.program_id(0)

    @pl.when(b == 0)
    def _():
        gg = jnp.maximum(g_ref[...], 0.0)
        c_ref[...] = lax.dot_general(
            w1_ref[...], gg, (((0,), (1,)), ((), ())),
            preferred_element_type=jnp.float32) + bl_ref[...]

    u_ref[...] = jnp.dot(x_ref[...], w2_ref[...],
                         preferred_element_type=jnp.float32)


def _sc_body(x3, u2, seg2, c_hbm, S_out, d_out,
             seg_v, u_v, cseg_v, e_v, xb, zbuf, zd_v, S_sh, d_sh,
             sem_g, sem_in, sem_s, sem_d):
    cid = lax.axis_index("c")
    sid = lax.axis_index("s")
    wid = sid * 2 + cid

    def _zrow(i, carry):
        for j in range(_F // 16):
            zbuf[i, pl.ds(j * 16, 16)] = jnp.zeros((16,), jnp.float32)
        return carry
    lax.fori_loop(0, _F, _zrow, 0)

    def _zd(i, carry):
        zd_v[pl.ds(i * 16, 16)] = jnp.zeros((16,), jnp.float32)
        return carry
    lax.fori_loop(0, _G // 16, _zd, 0)

    stripe = pl.multiple_of(sid * _F, _F)
    pltpu.sync_copy(zbuf, S_sh.at[pl.ds(stripe, _F)])

    @pl.when(sid == 0)
    def _():
        pltpu.sync_copy(zd_v, d_sh)

    plsc.subcore_barrier()

    @pl.when(wid < _NT)
    def _work():
        wsafe = jnp.minimum(wid, _NT - 1)
        base3 = wsafe * _RPT
        pltpu.sync_copy(seg2.at[wsafe], seg_v)
        pltpu.sync_copy(u2.at[wsafe], u_v)

        descs = [
            pltpu.async_copy(c_hbm.at[seg_v.at[r]], cseg_v.at[r], sem_g)
            for r in range(_RPT)
        ]
        for dsc in descs:
            dsc.wait()

        def _ebody(r, carry):
            for j in range(_CH // 16):
                sl = pl.ds(j * 16, 16)
                z = u_v[r, sl] + cseg_v[r, sl]
                z = jnp.maximum(z, 0.01 * z)
                z = jnp.minimum(z, 80.0)
                e_v[pl.ds(r * _CH + j * 16, 16)] = jnp.exp(z)
            return carry
        lax.fori_loop(0, _RPT, _ebody, 0)

        pltpu.async_copy(x3.at[base3], xb.at[0], sem_in)

        def _chunk(r, carry):
            cb = lax.rem(r, 2)

            @pl.when(r + 1 < _RPT)
            def _():
                pltpu.async_copy(x3.at[base3 + r + 1],
                                 xb.at[lax.rem(r + 1, 2)], sem_in)

            pltpu.make_async_copy(x3.at[base3], xb.at[0], sem_in).wait()

            rbase = r * _CH
            for g in range(_CH // 16):
                ev16 = e_v[pl.ds(rbase + g * 16, 16)]
                for k in range(16):
                    i = g * 16 + k
                    ev = _bcast_lane(ev16, k)
                    for j in range(_F // 16):
                        sl = pl.ds(j * 16, 16)
                        xb[cb, i, sl] = xb[cb, i, sl] * ev
            pltpu.sync_copy(xb.at[cb], S_sh.at[seg_v.at[r]], add=True)
            pltpu.sync_copy(e_v.at[pl.ds(rbase, _CH)],
                            d_sh.at[seg_v.at[r]], add=True)
            return carry
        lax.fori_loop(0, _RPT, _chunk, 0)

    plsc.subcore_barrier()

    pltpu.sync_copy(S_sh.at[pl.ds(stripe, _F)],
                    S_out.at[cid, pl.ds(stripe, _F)])

    @pl.when(sid == 0)
    def _():
        dslot = pl.multiple_of(cid * _G, _G)
        pltpu.sync_copy(d_sh, d_out.at[pl.ds(dslot, _G)])


def _gru_body(S_ref, d_ref, g_ref, Wp_ref, bp_ref, Wih_ref, Whh_ref,
              bih_ref, bhh_ref, out_ref):
    S = S_ref[0] + S_ref[1]
    d = d_ref[0] + d_ref[1]
    mask = d > 0.0
    inv = jnp.where(mask, 1.0 / jnp.where(mask, d, 1.0), 0.0)
    M = S * inv
    g = g_ref[...]
    grepr = lax.dot_general(M, Wp_ref[...], (((1,), (1,)), ((), ())),
                            preferred_element_type=jnp.float32)
    grepr = grepr + jnp.where(mask, bp_ref[...], 0.0)
    ctx = jnp.where(grepr > 0.0,
                    grepr,
                    jnp.exp(jnp.minimum(grepr, 0.0)) - 1.0)
    gi = lax.dot_general(ctx, Wih_ref[...], (((1,), (1,)), ((), ())),
                         preferred_element_type=jnp.float32) + bih_ref[...]
    gh = lax.dot_general(g, Whh_ref[...], (((1,), (1,)), ((), ())),
                         preferred_element_type=jnp.float32) + bhh_ref[...]
    F = g.shape[1]
    i_r, i_z, i_n = gi[:, :F], gi[:, F:2 * F], gi[:, 2 * F:]
    h_r, h_z, h_n = gh[:, :F], gh[:, F:2 * F], gh[:, 2 * F:]
    r = 1.0 / (1.0 + jnp.exp(-(i_r + h_r)))
    zg = 1.0 / (1.0 + jnp.exp(-(i_z + h_z)))
    n = jnp.tanh(i_n + r * h_n)
    out_ref[...] = (1.0 - zg) * n + zg * g


def kernel(node_feats, g_feats, segment_ids, W_logits, b_logits,
           W_proj, b_proj, W_ih, W_hh, b_ih, b_hh):
    V, F = node_feats.shape
    G = g_feats.shape[0]
    B = 2048
    nb = pl.cdiv(V, B)
    Vp = nb * B
    w1 = W_logits[:, :F].reshape(F, 1)
    w2 = W_logits[:, F:].reshape(F, 1)
    bl = b_logits.reshape(1, 1)

    u, c = pl.pallas_call(
        _u_body,
        grid=(nb,),
        in_specs=[
            pl.BlockSpec((B, F), lambda b: (b, 0)),
            pl.BlockSpec((G, F), lambda b: (0, 0)),
            pl.BlockSpec((F, 1), lambda b: (0, 0)),
            pl.BlockSpec((F, 1), lambda b: (0, 0)),
            pl.BlockSpec((1, 1), lambda b: (0, 0)),
        ],
        out_specs=[
            pl.BlockSpec((B, 1), lambda b: (b, 0)),
            pl.BlockSpec((1, G), lambda b: (0, 0)),
        ],
        out_shape=[
            jax.ShapeDtypeStruct((Vp, 1), jnp.float32),
            jax.ShapeDtypeStruct((1, G), jnp.float32),
        ],
    )(node_feats, g_feats, w1, w2, bl)

    u2d = u[:V, 0].reshape(_NT, _RPT, _CH)
    seg2d = segment_ids.reshape(_NT, _RPT, _CH)
    x3 = node_feats.reshape(_NROW, _CH, F)
    cflat = c.reshape(G)

    mesh = plsc.VectorSubcoreMesh(core_axis_name="c", subcore_axis_name="s")
    Spart, dflat = pl.kernel(
        _sc_body,
        out_type=[
            jax.ShapeDtypeStruct((2, G, F), jnp.float32),
            jax.ShapeDtypeStruct((2 * G,), jnp.float32),
        ],
        mesh=mesh,
        scratch_types=[
            pltpu.VMEM((_RPT, _CH), jnp.int32),
            pltpu.VMEM((_RPT, _CH), jnp.float32),
            pltpu.VMEM((_RPT, _CH), jnp.float32),
            pltpu.VMEM((_RPT * _CH,), jnp.float32),
            pltpu.VMEM((2, _CH, F), jnp.float32),
            pltpu.VMEM((F, F), jnp.float32),
            pltpu.VMEM((G,), jnp.float32),
            pltpu.VMEM_SHARED((G, F), jnp.float32),
            pltpu.VMEM_SHARED((G,), jnp.float32),
            pltpu.SemaphoreType.DMA,
            pltpu.SemaphoreType.DMA,
            pltpu.SemaphoreType.DMA,
            pltpu.SemaphoreType.DMA,
        ],
    )(x3, u2d, seg2d, cflat)

    Bg = 512
    ng = G // Bg
    bp = b_proj.reshape(1, F)
    bih = b_ih.reshape(1, 3 * F)
    bhh = b_hh.reshape(1, 3 * F)
    d3 = dflat.reshape(2, G, 1)

    out = pl.pallas_call(
        _gru_body,
        grid=(ng,),
        in_specs=[
            pl.BlockSpec((2, Bg, F), lambda b: (0, b, 0)),
            pl.BlockSpec((2, Bg, 1), lambda b: (0, b, 0)),
            pl.BlockSpec((Bg, F), lambda b: (b, 0)),
            pl.BlockSpec((F, F), lambda b: (0, 0)),
            pl.BlockSpec((1, F), lambda b: (0, 0)),
            pl.BlockSpec((3 * F, F), lambda b: (0, 0)),
            pl.BlockSpec((3 * F, F), lambda b: (0, 0)),
            pl.BlockSpec((1, 3 * F), lambda b: (0, 0)),
            pl.BlockSpec((1, 3 * F), lambda b: (0, 0)),
        ],
        out_specs=pl.BlockSpec((Bg, F), lambda b: (b, 0)),
        out_shape=jax.ShapeDtypeStruct((G, F), jnp.float32),
    )(Spart, d3, g_feats, W_proj, bp, W_ih, W_hh, bih, bhh)
    return out


# speedup vs baseline: 1.5737x; 1.5737x over previous
import functools

import jax
import jax.numpy as jnp
from jax import lax
from jax.experimental import pallas as pl
from jax.experimental.pallas import tpu as pltpu
from jax.experimental.pallas import tpu_sc as plsc

_BCAST_DNUMS = lax.GatherDimensionNumbers(
    offset_dims=(), collapsed_slice_dims=(0,), start_index_map=(0,))


def _bcast_lane(vec16, k):
    idx = jnp.full((16, 1), k, jnp.int32)
    return lax.gather(vec16, idx, _BCAST_DNUMS, (1,),
                      mode=lax.GatherScatterMode.PROMISE_IN_BOUNDS)


_G = 2048
_F = 128
_V = 100000
_CH = 80
_RPT = 50
_NT = 25
_NROW = _V // _CH


def _u_body(x_ref, g_ref, w1_ref, w2_ref, bl_ref, u_ref, c_ref):
    b = pl.program_id(0)

    @pl.when(b == 0)
    def _():
        gg = jnp.maximum(g_ref[...], 0.0)
        c_ref[...] = lax.dot_general(
            w1_ref[...], gg, (((0,), (1,)), ((), ())),
            preferred_element_type=jnp.float32) + bl_ref[...]

    u_ref[...] = jnp.dot(x_ref[...], w2_ref[...],
                         preferred_element_type=jnp.float32)


def _sc_body(x_hbm, u_hbm, seg_hbm, c_hbm, S_out, d_out,
             seg2d, u_fl, seg_fl, cseg_v, e_v, xb, zbuf, zd_v, S_sh, d_sh,
             sem_g, sem_in, sem_s, sem_d):
    cid = lax.axis_index("c")
    sid = lax.axis_index("s")
    wid = sid * 2 + cid

    def _zrow(i, carry):
        for j in range(_F // 16):
            zbuf[i, pl.ds(j * 16, 16)] = jnp.zeros((16,), jnp.float32)
        return carry
    lax.fori_loop(0, _F, _zrow, 0)

    def _zd(i, carry):
        zd_v[pl.ds(i * 16, 16)] = jnp.zeros((16,), jnp.float32)
        return carry
    lax.fori_loop(0, _G // 16, _zd, 0)

    stripe = pl.multiple_of(sid * _F, _F)
    pltpu.sync_copy(zbuf, S_sh.at[pl.ds(stripe, _F)])

    @pl.when(sid == 0)
    def _():
        pltpu.sync_copy(zd_v, d_sh)

    plsc.subcore_barrier()

    @pl.when(wid < _NT)
    def _work():
        wsafe = jnp.minimum(wid, _NT - 1)
        npt = _RPT * _CH
        nbase = pl.multiple_of(wsafe * npt, 16)
        pltpu.sync_copy(u_hbm.at[pl.ds(nbase, npt)], u_fl)
        pltpu.sync_copy(seg_hbm.at[pl.ds(nbase, npt)], seg_fl)

        def _rp(r, carry):
            for j in range(_CH // 16):
                seg2d[r, pl.ds(j * 16, 16)] = (
                    seg_fl[pl.ds(r * _CH + j * 16, 16)])
            return carry
        lax.fori_loop(0, _RPT, _rp, 0)

        descs = [
            pltpu.async_copy(c_hbm.at[seg2d.at[r]], cseg_v.at[r], sem_g)
            for r in range(_RPT)
        ]
        for dsc in descs:
            dsc.wait()

        def _ebody(r, carry):
            for j in range(_CH // 16):
                fsl = pl.ds(r * _CH + j * 16, 16)
                z = u_fl[fsl] + cseg_v[r, pl.ds(j * 16, 16)]
                z = jnp.maximum(z, 0.01 * z)
                z = jnp.minimum(z, 80.0)
                e_v[fsl] = jnp.exp(z)
            return carry
        lax.fori_loop(0, _RPT, _ebody, 0)

        def _xsl(r):
            return x_hbm.at[pl.ds(pl.multiple_of(nbase + r * _CH, 16),
                                  _CH), :]

        pltpu.async_copy(_xsl(0), xb.at[0], sem_in)

        def _chunk(r, carry):
            cb = lax.rem(r, 3)

            @pl.when(r >= 2)
            def _():
                pltpu.make_async_copy(_xsl(0), xb.at[0], sem_s).wait()

            @pl.when(r + 1 < _RPT)
            def _():
                pltpu.async_copy(_xsl(r + 1), xb.at[lax.rem(r + 1, 3)],
                                 sem_in)

            pltpu.make_async_copy(_xsl(0), xb.at[0], sem_in).wait()

            xrow = xb.at[cb]
            rbase = r * _CH
            for g in range(_CH // 16):
                ev16 = e_v[pl.ds(rbase + g * 16, 16)]
                for k in range(16):
                    i = g * 16 + k
                    ev = _bcast_lane(ev16, k)
                    for j in range(_F // 16):
                        sl = pl.ds(j * 16, 16)
                        xrow[i, sl] = xrow[i, sl] * ev
            pltpu.async_copy(xb.at[cb], S_sh.at[seg2d.at[r]], sem_s,
                             add=True)
            pltpu.async_copy(e_v.at[pl.ds(rbase, _CH)],
                             d_sh.at[seg2d.at[r]], sem_d, add=True)

            @pl.when(r >= 10)
            def _():
                pltpu.make_async_copy(u_hbm.at[pl.ds(nbase, _CH)],
                                      e_v.at[pl.ds(0, _CH)], sem_d).wait()
            return carry
        lax.fori_loop(0, _RPT, _chunk, 0)

        for _ in range(2):
            pltpu.make_async_copy(_xsl(0), xb.at[0], sem_s).wait()
        for _ in range(10):
            pltpu.make_async_copy(u_hbm.at[pl.ds(nbase, _CH)],
                                  e_v.at[pl.ds(0, _CH)], sem_d).wait()

    plsc.subcore_barrier()

    pltpu.sync_copy(S_sh.at[pl.ds(stripe, _F)],
                    S_out.at[cid, pl.ds(stripe, _F)])

    @pl.when(sid == 0)
    def _():
        dslot = pl.multiple_of(cid * _G, _G)
        pltpu.sync_copy(d_sh, d_out.at[pl.ds(dslot, _G)])


def _gru_body(S_ref, d_ref, g_ref, Wp_ref, bp_ref, Wih_ref, Whh_ref,
              bih_ref, bhh_ref, out_ref):
    S = S_ref[0] + S_ref[1]
    d = d_ref[0] + d_ref[1]
    mask = d > 0.0
    inv = jnp.where(mask, 1.0 / jnp.where(mask, d, 1.0), 0.0)
    M = S * inv
    g = g_ref[...]
    grepr = lax.dot_general(M, Wp_ref[...], (((1,), (1,)), ((), ())),
                            preferred_element_type=jnp.float32)
    grepr = grepr + jnp.where(mask, bp_ref[...], 0.0)
    ctx = jnp.where(grepr > 0.0,
                    grepr,
                    jnp.exp(jnp.minimum(grepr, 0.0)) - 1.0)
    gi = lax.dot_general(ctx, Wih_ref[...], (((1,), (1,)), ((), ())),
                         preferred_element_type=jnp.float32) + bih_ref[...]
    gh = lax.dot_general(g, Whh_ref[...], (((1,), (1,)), ((), ())),
                         preferred_element_type=jnp.float32) + bhh_ref[...]
    F = g.shape[1]
    i_r, i_z, i_n = gi[:, :F], gi[:, F:2 * F], gi[:, 2 * F:]
    h_r, h_z, h_n = gh[:, :F], gh[:, F:2 * F], gh[:, 2 * F:]
    r = 1.0 / (1.0 + jnp.exp(-(i_r + h_r)))
    zg = 1.0 / (1.0 + jnp.exp(-(i_z + h_z)))
    n = jnp.tanh(i_n + r * h_n)
    out_ref[...] = (1.0 - zg) * n + zg * g


def kernel(node_feats, g_feats, segment_ids, W_logits, b_logits,
           W_proj, b_proj, W_ih, W_hh, b_ih, b_hh):
    V, F = node_feats.shape
    G = g_feats.shape[0]
    B = 2048
    nb = pl.cdiv(V, B)
    Vp = nb * B
    w1 = W_logits[:, :F].reshape(F, 1)
    w2 = W_logits[:, F:].reshape(F, 1)
    bl = b_logits.reshape(1, 1)

    u, c = pl.pallas_call(
        _u_body,
        grid=(nb,),
        in_specs=[
            pl.BlockSpec((B, F), lambda b: (b, 0)),
            pl.BlockSpec((G, F), lambda b: (0, 0)),
            pl.BlockSpec((F, 1), lambda b: (0, 0)),
            pl.BlockSpec((F, 1), lambda b: (0, 0)),
            pl.BlockSpec((1, 1), lambda b: (0, 0)),
        ],
        out_specs=[
            pl.BlockSpec((B, 1), lambda b: (b, 0)),
            pl.BlockSpec((1, G), lambda b: (0, 0)),
        ],
        out_shape=[
            jax.ShapeDtypeStruct((Vp, 1), jnp.float32),
            jax.ShapeDtypeStruct((1, G), jnp.float32),
        ],
    )(node_feats, g_feats, w1, w2, bl)

    u1d = u.reshape(Vp)
    cflat = c.reshape(G)

    mesh = plsc.VectorSubcoreMesh(core_axis_name="c", subcore_axis_name="s")
    Spart, dflat = pl.kernel(
        _sc_body,
        out_type=[
            jax.ShapeDtypeStruct((2, G, F), jnp.float32),
            jax.ShapeDtypeStruct((2 * G,), jnp.float32),
        ],
        mesh=mesh,
        scratch_types=[
            pltpu.VMEM((_RPT, _CH), jnp.int32),
            pltpu.VMEM((_RPT * _CH,), jnp.float32),
            pltpu.VMEM((_RPT * _CH,), jnp.int32),
            pltpu.VMEM((_RPT, _CH), jnp.float32),
            pltpu.VMEM((_RPT * _CH,), jnp.float32),
            pltpu.VMEM((3, _CH, F), jnp.float32),
            pltpu.VMEM((F, F), jnp.float32),
            pltpu.VMEM((G,), jnp.float32),
            pltpu.VMEM_SHARED((G, F), jnp.float32),
            pltpu.VMEM_SHARED((G,), jnp.float32),
            pltpu.SemaphoreType.DMA,
            pltpu.SemaphoreType.DMA,
            pltpu.SemaphoreType.DMA,
            pltpu.SemaphoreType.DMA,
        ],
    )(node_feats, u1d, segment_ids, cflat)

    Bg = 512
    ng = G // Bg
    bp = b_proj.reshape(1, F)
    bih = b_ih.reshape(1, 3 * F)
    bhh = b_hh.reshape(1, 3 * F)
    d3 = dflat.reshape(2, G, 1)

    out = pl.pallas_call(
        _gru_body,
        grid=(ng,),
        in_specs=[
            pl.BlockSpec((2, Bg, F), lambda b: (0, b, 0)),
            pl.BlockSpec((2, Bg, 1), lambda b: (0, b, 0)),
            pl.BlockSpec((Bg, F), lambda b: (b, 0)),
            pl.BlockSpec((F, F), lambda b: (0, 0)),
            pl.BlockSpec((1, F), lambda b: (0, 0)),
            pl.BlockSpec((3 * F, F), lambda b: (0, 0)),
            pl.BlockSpec((3 * F, F), lambda b: (0, 0)),
            pl.BlockSpec((1, 3 * F), lambda b: (0, 0)),
            pl.BlockSpec((1, 3 * F), lambda b: (0, 0)),
        ],
        out_specs=pl.BlockSpec((Bg, F), lambda b: (b, 0)),
        out_shape=jax.ShapeDtypeStruct((G, F), jnp.float32),
    )(Spart, d3, g_feats, W_proj, bp, W_ih, W_hh, bih, bhh)
    return out


# speedup vs baseline: 1.6576x; 1.0533x over previous
import functools

import jax
import jax.numpy as jnp
from jax import lax
from jax.experimental import pallas as pl
from jax.experimental.pallas import tpu as pltpu
from jax.experimental.pallas import tpu_sc as plsc

_BCAST_DNUMS = lax.GatherDimensionNumbers(
    offset_dims=(), collapsed_slice_dims=(0,), start_index_map=(0,))


def _bcast_lane(vec16, k):
    idx = jnp.full((16, 1), k, jnp.int32)
    return lax.gather(vec16, idx, _BCAST_DNUMS, (1,),
                      mode=lax.GatherScatterMode.PROMISE_IN_BOUNDS)


_G = 2048
_F = 128
_V = 100000
_CH = 80
_NCHK = _V // _CH
_NW = 32
_RPTM = 40


def _u_body(x_ref, g_ref, w1_ref, w2_ref, bl_ref, u_ref, c_ref):
    b = pl.program_id(0)

    @pl.when(b == 0)
    def _():
        gg = jnp.maximum(g_ref[...], 0.0)
        c_ref[...] = lax.dot_general(
            w1_ref[...], gg, (((0,), (1,)), ((), ())),
            preferred_element_type=jnp.float32) + bl_ref[...]

    u_ref[...] = jnp.dot(x_ref[...], w2_ref[...],
                         preferred_element_type=jnp.float32)


def _sc_body(x_hbm, u_hbm, seg_hbm, c_hbm, S_out, d_out,
             seg2d, u_fl, seg_fl, cseg_v, e_v, xb, zbuf, zd_v, S_sh, d_sh,
             sem_g, sem_in, sem_s, sem_d):
    cid = lax.axis_index("c")
    sid = lax.axis_index("s")
    wid = sid * 2 + cid

    def _zrow(i, carry):
        for j in range(_F // 16):
            zbuf[i, pl.ds(j * 16, 16)] = jnp.zeros((16,), jnp.float32)
        return carry
    lax.fori_loop(0, _F, _zrow, 0)

    def _zd(i, carry):
        zd_v[pl.ds(i * 16, 16)] = jnp.zeros((16,), jnp.float32)
        return carry
    lax.fori_loop(0, _G // 16, _zd, 0)

    stripe = pl.multiple_of(sid * _F, _F)
    pltpu.sync_copy(zbuf, S_sh.at[pl.ds(stripe, _F)])

    @pl.when(sid == 0)
    def _():
        pltpu.sync_copy(zd_v, d_sh)

    plsc.subcore_barrier()

    ncht = jnp.where(wid < 2, _RPTM, _RPTM - 1)
    cstart = wid * (_RPTM - 1) + jnp.minimum(wid, 2)
    nbase = pl.multiple_of(cstart * _CH, 16)
    npt = _RPTM * _CH
    pltpu.sync_copy(u_hbm.at[pl.ds(nbase, npt)], u_fl)
    pltpu.sync_copy(seg_hbm.at[pl.ds(nbase, npt)], seg_fl)

    def _rp(r, carry):
        for j in range(_CH // 16):
            seg2d[r, pl.ds(j * 16, 16)] = (
                seg_fl[pl.ds(r * _CH + j * 16, 16)])
        return carry
    lax.fori_loop(0, ncht, _rp, 0)

    descs = [
        pltpu.async_copy(c_hbm.at[seg2d.at[r]], cseg_v.at[r], sem_g)
        for r in range(_RPTM - 1)
    ]

    @pl.when(ncht == _RPTM)
    def _():
        pltpu.async_copy(c_hbm.at[seg2d.at[_RPTM - 1]],
                         cseg_v.at[_RPTM - 1], sem_g)

    for dsc in descs:
        dsc.wait()

    @pl.when(ncht == _RPTM)
    def _():
        pltpu.make_async_copy(c_hbm.at[seg2d.at[_RPTM - 1]],
                              cseg_v.at[_RPTM - 1], sem_g).wait()

    def _ebody(r, carry):
        for j in range(_CH // 16):
            fsl = pl.ds(r * _CH + j * 16, 16)
            z = u_fl[fsl] + cseg_v[r, pl.ds(j * 16, 16)]
            z = jnp.maximum(z, 0.01 * z)
            z = jnp.minimum(z, 80.0)
            e_v[fsl] = jnp.exp(z)
        return carry
    lax.fori_loop(0, ncht, _ebody, 0)

    def _xsl(r):
        return x_hbm.at[pl.ds(pl.multiple_of(nbase + r * _CH, 16),
                              _CH), :]

    pltpu.async_copy(_xsl(0), xb.at[0], sem_in)

    def _chunk(r, carry):
        cb = lax.rem(r, 3)

        @pl.when(r >= 2)
        def _():
            pltpu.make_async_copy(_xsl(0), xb.at[0], sem_s).wait()

        @pl.when(r + 1 < ncht)
        def _():
            pltpu.async_copy(_xsl(r + 1), xb.at[lax.rem(r + 1, 3)],
                             sem_in)

        pltpu.make_async_copy(_xsl(0), xb.at[0], sem_in).wait()

        xrow = xb.at[cb]
        rbase = r * _CH
        for g in range(_CH // 16):
            ev16 = e_v[pl.ds(rbase + g * 16, 16)]
            for k in range(16):
                i = g * 16 + k
                ev = _bcast_lane(ev16, k)
                for j in range(_F // 16):
                    sl = pl.ds(j * 16, 16)
                    xrow[i, sl] = xrow[i, sl] * ev
        pltpu.async_copy(xb.at[cb], S_sh.at[seg2d.at[r]], sem_s,
                         add=True)
        pltpu.async_copy(e_v.at[pl.ds(rbase, _CH)],
                         d_sh.at[seg2d.at[r]], sem_d, add=True)

        @pl.when(r >= 10)
        def _():
            pltpu.make_async_copy(u_hbm.at[pl.ds(nbase, _CH)],
                                  e_v.at[pl.ds(0, _CH)], sem_d).wait()
        return carry
    lax.fori_loop(0, ncht, _chunk, 0)

    for _ in range(2):
        pltpu.make_async_copy(_xsl(0), xb.at[0], sem_s).wait()
    for _ in range(10):
        pltpu.make_async_copy(u_hbm.at[pl.ds(nbase, _CH)],
                              e_v.at[pl.ds(0, _CH)], sem_d).wait()

    plsc.subcore_barrier()

    pltpu.sync_copy(S_sh.at[pl.ds(stripe, _F)],
                    S_out.at[cid, pl.ds(stripe, _F)])

    @pl.when(sid == 0)
    def _():
        dslot = pl.multiple_of(cid * _G, _G)
        pltpu.sync_copy(d_sh, d_out.at[pl.ds(dslot, _G)])


def _gru_body(S_ref, d_ref, g_ref, Wp_ref, bp_ref, Wih_ref, Whh_ref,
              bih_ref, bhh_ref, out_ref):
    S = S_ref[0] + S_ref[1]
    d = d_ref[0] + d_ref[1]
    mask = d > 0.0
    inv = jnp.where(mask, 1.0 / jnp.where(mask, d, 1.0), 0.0)
    M = S * inv
    g = g_ref[...]
    grepr = lax.dot_general(M, Wp_ref[...], (((1,), (1,)), ((), ())),
                            preferred_element_type=jnp.float32)
    grepr = grepr + jnp.where(mask, bp_ref[...], 0.0)
    ctx = jnp.where(grepr > 0.0,
                    grepr,
                    jnp.exp(jnp.minimum(grepr, 0.0)) - 1.0)
    gi = lax.dot_general(ctx, Wih_ref[...], (((1,), (1,)), ((), ())),
                         preferred_element_type=jnp.float32) + bih_ref[...]
    gh = lax.dot_general(g, Whh_ref[...], (((1,), (1,)), ((), ())),
                         preferred_element_type=jnp.float32) + bhh_ref[...]
    F = g.shape[1]
    i_r, i_z, i_n = gi[:, :F], gi[:, F:2 * F], gi[:, 2 * F:]
    h_r, h_z, h_n = gh[:, :F], gh[:, F:2 * F], gh[:, 2 * F:]
    r = 1.0 / (1.0 + jnp.exp(-(i_r + h_r)))
    zg = 1.0 / (1.0 + jnp.exp(-(i_z + h_z)))
    n = jnp.tanh(i_n + r * h_n)
    out_ref[...] = (1.0 - zg) * n + zg * g


def kernel(node_feats, g_feats, segment_ids, W_logits, b_logits,
           W_proj, b_proj, W_ih, W_hh, b_ih, b_hh):
    V, F = node_feats.shape
    G = g_feats.shape[0]
    B = 2048
    nb = pl.cdiv(V, B)
    Vp = nb * B
    w1 = W_logits[:, :F].reshape(F, 1)
    w2 = W_logits[:, F:].reshape(F, 1)
    bl = b_logits.reshape(1, 1)

    u, c = pl.pallas_call(
        _u_body,
        grid=(nb,),
        in_specs=[
            pl.BlockSpec((B, F), lambda b: (b, 0)),
            pl.BlockSpec((G, F), lambda b: (0, 0)),
            pl.BlockSpec((F, 1), lambda b: (0, 0)),
            pl.BlockSpec((F, 1), lambda b: (0, 0)),
            pl.BlockSpec((1, 1), lambda b: (0, 0)),
        ],
        out_specs=[
            pl.BlockSpec((B, 1), lambda b: (b, 0)),
            pl.BlockSpec((1, G), lambda b: (0, 0)),
        ],
        out_shape=[
            jax.ShapeDtypeStruct((Vp, 1), jnp.float32),
            jax.ShapeDtypeStruct((1, G), jnp.float32),
        ],
    )(node_feats, g_feats, w1, w2, bl)

    u1d = u.reshape(Vp)
    cflat = c.reshape(G)
    segp = jnp.zeros((Vp,), segment_ids.dtype).at[:V].set(segment_ids)

    mesh = plsc.VectorSubcoreMesh(core_axis_name="c", subcore_axis_name="s")
    Spart, dflat = pl.kernel(
        _sc_body,
        out_type=[
            jax.ShapeDtypeStruct((2, G, F), jnp.float32),
            jax.ShapeDtypeStruct((2 * G,), jnp.float32),
        ],
        mesh=mesh,
        scratch_types=[
            pltpu.VMEM((_RPTM, _CH), jnp.int32),
            pltpu.VMEM((_RPTM * _CH,), jnp.float32),
            pltpu.VMEM((_RPTM * _CH,), jnp.int32),
            pltpu.VMEM((_RPTM, _CH), jnp.float32),
            pltpu.VMEM((_RPTM * _CH,), jnp.float32),
            pltpu.VMEM((3, _CH, F), jnp.float32),
            pltpu.VMEM((F, F), jnp.float32),
            pltpu.VMEM((G,), jnp.float32),
            pltpu.VMEM_SHARED((G, F), jnp.float32),
            pltpu.VMEM_SHARED((G,), jnp.float32),
            pltpu.SemaphoreType.DMA,
            pltpu.SemaphoreType.DMA,
            pltpu.SemaphoreType.DMA,
            pltpu.SemaphoreType.DMA,
        ],
    )(node_feats, u1d, segp, cflat)

    Bg = 512
    ng = G // Bg
    bp = b_proj.reshape(1, F)
    bih = b_ih.reshape(1, 3 * F)
    bhh = b_hh.reshape(1, 3 * F)
    d3 = dflat.reshape(2, G, 1)

    out = pl.pallas_call(
        _gru_body,
        grid=(ng,),
        in_specs=[
            pl.BlockSpec((2, Bg, F), lambda b: (0, b, 0)),
            pl.BlockSpec((2, Bg, 1), lambda b: (0, b, 0)),
            pl.BlockSpec((Bg, F), lambda b: (b, 0)),
            pl.BlockSpec((F, F), lambda b: (0, 0)),
            pl.BlockSpec((1, F), lambda b: (0, 0)),
            pl.BlockSpec((3 * F, F), lambda b: (0, 0)),
            pl.BlockSpec((3 * F, F), lambda b: (0, 0)),
            pl.BlockSpec((1, 3 * F), lambda b: (0, 0)),
            pl.BlockSpec((1, 3 * F), lambda b: (0, 0)),
        ],
        out_specs=pl.BlockSpec((Bg, F), lambda b: (b, 0)),
        out_shape=jax.ShapeDtypeStruct((G, F), jnp.float32),
    )(Spart, d3, g_feats, W_proj, bp, W_ih, W_hh, bih, bhh)
    return out


# speedup vs baseline: 1.8110x; 1.0926x over previous
import functools

import jax
import jax.numpy as jnp
from jax import lax
from jax.experimental import pallas as pl
from jax.experimental.pallas import tpu as pltpu
from jax.experimental.pallas import tpu_sc as plsc

_BCAST_DNUMS = lax.GatherDimensionNumbers(
    offset_dims=(), collapsed_slice_dims=(0,), start_index_map=(0,))


def _bcast_lane(vec16, k):
    idx = jnp.full((16, 1), k, jnp.int32)
    return lax.gather(vec16, idx, _BCAST_DNUMS, (1,),
                      mode=lax.GatherScatterMode.PROMISE_IN_BOUNDS)


_G = 2048
_F = 128
_V = 100000
_CH = 80
_NCHK = _V // _CH
_NW = 32
_RPTM = 40


def _u_body(x_ref, g_ref, w1_ref, w2_ref, bl_ref, u_ref, c_ref):
    b = pl.program_id(0)

    @pl.when(b == 0)
    def _():
        gg = jnp.maximum(g_ref[...], 0.0)
        c_ref[...] = lax.dot_general(
            w1_ref[...], gg, (((0,), (1,)), ((), ())),
            preferred_element_type=jnp.float32) + bl_ref[...]

    u_ref[...] = jnp.dot(x_ref[...], w2_ref[...],
                         preferred_element_type=jnp.float32)


def _sc_body(x_hbm, u_hbm, seg_hbm, c_hbm, S_out, d_out,
             seg2d, u_fl, seg_fl, cseg_v, e_v, xb, zbuf, zd_v, S_sh, d_sh,
             sem_g, sem_in, sem_s, sem_d):
    cid = lax.axis_index("c")
    sid = lax.axis_index("s")
    wid = sid * 2 + cid

    def _zrow(i, carry):
        for j in range(_F // 16):
            zbuf[i, pl.ds(j * 16, 16)] = jnp.zeros((16,), jnp.float32)
        return carry
    lax.fori_loop(0, _F, _zrow, 0)

    def _zd(i, carry):
        zd_v[pl.ds(i * 16, 16)] = jnp.zeros((16,), jnp.float32)
        return carry
    lax.fori_loop(0, _G // 16, _zd, 0)

    stripe = pl.multiple_of(sid * _F, _F)
    pltpu.sync_copy(zbuf, S_sh.at[pl.ds(stripe, _F)])

    @pl.when(sid == 0)
    def _():
        pltpu.sync_copy(zd_v, d_sh)

    plsc.subcore_barrier()

    ncht = jnp.where(wid < 2, _RPTM, _RPTM - 1)
    cstart = wid * (_RPTM - 1) + jnp.minimum(wid, 2)
    nbase = pl.multiple_of(cstart * _CH, 16)
    npt = _RPTM * _CH
    pltpu.sync_copy(u_hbm.at[pl.ds(nbase, npt)], u_fl)
    pltpu.sync_copy(seg_hbm.at[pl.ds(nbase, npt)], seg_fl)

    def _rp(r, carry):
        for j in range(_CH // 16):
            seg2d[r, pl.ds(j * 16, 16)] = (
                seg_fl[pl.ds(r * _CH + j * 16, 16)])
        return carry
    lax.fori_loop(0, ncht, _rp, 0)

    descs = [
        pltpu.async_copy(c_hbm.at[seg2d.at[r]], cseg_v.at[r], sem_g)
        for r in range(_RPTM - 1)
    ]

    @pl.when(ncht == _RPTM)
    def _():
        pltpu.async_copy(c_hbm.at[seg2d.at[_RPTM - 1]],
                         cseg_v.at[_RPTM - 1], sem_g)

    for dsc in descs:
        dsc.wait()

    @pl.when(ncht == _RPTM)
    def _():
        pltpu.make_async_copy(c_hbm.at[seg2d.at[_RPTM - 1]],
                              cseg_v.at[_RPTM - 1], sem_g).wait()

    def _ebody(r, carry):
        for j in range(_CH // 16):
            fsl = pl.ds(r * _CH + j * 16, 16)
            z = u_fl[fsl] + cseg_v[r, pl.ds(j * 16, 16)]
            z = jnp.maximum(z, 0.01 * z)
            z = jnp.minimum(z, 80.0)
            e_v[fsl] = jnp.exp(z)
        return carry
    lax.fori_loop(0, ncht, _ebody, 0)

    def _xsl(r):
        return x_hbm.at[pl.ds(pl.multiple_of(nbase + r * _CH, 16),
                              _CH), :]

    pltpu.async_copy(_xsl(0), xb.at[0], sem_in)

    def _chunk(r, carry):
        cb = lax.rem(r, 3)

        @pl.when(r >= 2)
        def _():
            pltpu.make_async_copy(_xsl(0), xb.at[0], sem_s).wait()

        @pl.when(r + 1 < ncht)
        def _():
            pltpu.async_copy(_xsl(r + 1), xb.at[lax.rem(r + 1, 3)],
                             sem_in)

        pltpu.make_async_copy(_xsl(0), xb.at[0], sem_in).wait()

        xrow = xb.at[cb]
        rbase = r * _CH
        for g in range(_CH // 16):
            ev16 = e_v[pl.ds(rbase + g * 16, 16)]
            for k in range(16):
                i = g * 16 + k
                ev = _bcast_lane(ev16, k)
                for j in range(_F // 16):
                    sl = pl.ds(j * 16, 16)
                    xrow[i, sl] = xrow[i, sl] * ev
        pltpu.async_copy(xb.at[cb], S_sh.at[seg2d.at[r]], sem_s,
                         add=True)
        pltpu.async_copy(e_v.at[pl.ds(rbase, _CH)],
                         d_sh.at[seg2d.at[r]], sem_d, add=True)

        @pl.when(r >= 10)
        def _():
            pltpu.make_async_copy(u_hbm.at[pl.ds(nbase, _CH)],
                                  e_v.at[pl.ds(0, _CH)], sem_d).wait()
        return carry
    lax.fori_loop(0, ncht, _chunk, 0)

    for _ in range(2):
        pltpu.make_async_copy(_xsl(0), xb.at[0], sem_s).wait()
    for _ in range(10):
        pltpu.make_async_copy(u_hbm.at[pl.ds(nbase, _CH)],
                              e_v.at[pl.ds(0, _CH)], sem_d).wait()

    plsc.subcore_barrier()

    pltpu.sync_copy(S_sh.at[pl.ds(stripe, _F)],
                    S_out.at[cid, pl.ds(stripe, _F)])

    @pl.when(sid == 0)
    def _():
        dslot = pl.multiple_of(cid * _G, _G)
        pltpu.sync_copy(d_sh, d_out.at[pl.ds(dslot, _G)])


def _gru_body(S_ref, d_ref, g_ref, Wp_ref, bp_ref, Wih_ref, Whh_ref,
              bih_ref, bhh_ref, out_ref):
    S = S_ref[0] + S_ref[1]
    d = d_ref[0] + d_ref[1]
    mask = d > 0.0
    inv = jnp.where(mask, 1.0 / jnp.where(mask, d, 1.0), 0.0)
    M = S * inv
    g = g_ref[...]
    grepr = lax.dot_general(M, Wp_ref[...], (((1,), (1,)), ((), ())),
                            preferred_element_type=jnp.float32)
    grepr = grepr + jnp.where(mask, bp_ref[...], 0.0)
    ctx = jnp.where(grepr > 0.0,
                    grepr,
                    jnp.exp(jnp.minimum(grepr, 0.0)) - 1.0)
    gi = lax.dot_general(ctx, Wih_ref[...], (((1,), (1,)), ((), ())),
                         preferred_element_type=jnp.float32) + bih_ref[...]
    gh = lax.dot_general(g, Whh_ref[...], (((1,), (1,)), ((), ())),
                         preferred_element_type=jnp.float32) + bhh_ref[...]
    F = g.shape[1]
    i_r, i_z, i_n = gi[:, :F], gi[:, F:2 * F], gi[:, 2 * F:]
    h_r, h_z, h_n = gh[:, :F], gh[:, F:2 * F], gh[:, 2 * F:]
    r = 1.0 / (1.0 + jnp.exp(-(i_r + h_r)))
    zg = 1.0 / (1.0 + jnp.exp(-(i_z + h_z)))
    n = jnp.tanh(i_n + r * h_n)
    out_ref[...] = (1.0 - zg) * n + zg * g


def kernel(node_feats, g_feats, segment_ids, W_logits, b_logits,
           W_proj, b_proj, W_ih, W_hh, b_ih, b_hh):
    V, F = node_feats.shape
    G = g_feats.shape[0]
    B = 4096
    nb = pl.cdiv(V, B)
    Vp = nb * B
    w1 = W_logits[:, :F].reshape(F, 1)
    w2 = W_logits[:, F:].reshape(F, 1)
    bl = b_logits.reshape(1, 1)

    u, c = pl.pallas_call(
        _u_body,
        grid=(nb,),
        in_specs=[
            pl.BlockSpec((B, F), lambda b: (b, 0)),
            pl.BlockSpec((G, F), lambda b: (0, 0)),
            pl.BlockSpec((F, 1), lambda b: (0, 0)),
            pl.BlockSpec((F, 1), lambda b: (0, 0)),
            pl.BlockSpec((1, 1), lambda b: (0, 0)),
        ],
        out_specs=[
            pl.BlockSpec((B, 1), lambda b: (b, 0)),
            pl.BlockSpec((1, G), lambda b: (0, 0)),
        ],
        out_shape=[
            jax.ShapeDtypeStruct((Vp, 1), jnp.float32),
            jax.ShapeDtypeStruct((1, G), jnp.float32),
        ],
    )(node_feats, g_feats, w1, w2, bl)

    u1d = u.reshape(Vp)
    cflat = c.reshape(G)
    segp = jnp.zeros((Vp,), segment_ids.dtype).at[:V].set(segment_ids)

    mesh = plsc.VectorSubcoreMesh(core_axis_name="c", subcore_axis_name="s")
    Spart, dflat = pl.kernel(
        _sc_body,
        out_type=[
            jax.ShapeDtypeStruct((2, G, F), jnp.float32),
            jax.ShapeDtypeStruct((2 * G,), jnp.float32),
        ],
        mesh=mesh,
        scratch_types=[
            pltpu.VMEM((_RPTM, _CH), jnp.int32),
            pltpu.VMEM((_RPTM * _CH,), jnp.float32),
            pltpu.VMEM((_RPTM * _CH,), jnp.int32),
            pltpu.VMEM((_RPTM, _CH), jnp.float32),
            pltpu.VMEM((_RPTM * _CH,), jnp.float32),
            pltpu.VMEM((3, _CH, F), jnp.float32),
            pltpu.VMEM((F, F), jnp.float32),
            pltpu.VMEM((G,), jnp.float32),
            pltpu.VMEM_SHARED((G, F), jnp.float32),
            pltpu.VMEM_SHARED((G,), jnp.float32),
            pltpu.SemaphoreType.DMA,
            pltpu.SemaphoreType.DMA,
            pltpu.SemaphoreType.DMA,
            pltpu.SemaphoreType.DMA,
        ],
    )(node_feats, u1d, segp, cflat)

    Bg = 512
    ng = G // Bg
    bp = b_proj.reshape(1, F)
    bih = b_ih.reshape(1, 3 * F)
    bhh = b_hh.reshape(1, 3 * F)
    d3 = dflat.reshape(2, G, 1)

    out = pl.pallas_call(
        _gru_body,
        grid=(ng,),
        in_specs=[
            pl.BlockSpec((2, Bg, F), lambda b: (0, b, 0)),
            pl.BlockSpec((2, Bg, 1), lambda b: (0, b, 0)),
            pl.BlockSpec((Bg, F), lambda b: (b, 0)),
            pl.BlockSpec((F, F), lambda b: (0, 0)),
            pl.BlockSpec((1, F), lambda b: (0, 0)),
            pl.BlockSpec((3 * F, F), lambda b: (0, 0)),
            pl.BlockSpec((3 * F, F), lambda b: (0, 0)),
            pl.BlockSpec((1, 3 * F), lambda b: (0, 0)),
            pl.BlockSpec((1, 3 * F), lambda b: (0, 0)),
        ],
        out_specs=pl.BlockSpec((Bg, F), lambda b: (b, 0)),
        out_shape=jax.ShapeDtypeStruct((G, F), jnp.float32),
    )(Spart, d3, g_feats, W_proj, bp, W_ih, W_hh, bih, bhh)
    return out


# speedup vs baseline: 1.8524x; 1.0228x over previous
import functools

import jax
import jax.numpy as jnp
from jax import lax
from jax.experimental import pallas as pl
from jax.experimental.pallas import tpu as pltpu
from jax.experimental.pallas import tpu_sc as plsc

_BCAST_DNUMS = lax.GatherDimensionNumbers(
    offset_dims=(), collapsed_slice_dims=(0,), start_index_map=(0,))


def _bcast_lane(vec16, k):
    idx = jnp.full((16, 1), k, jnp.int32)
    return lax.gather(vec16, idx, _BCAST_DNUMS, (1,),
                      mode=lax.GatherScatterMode.PROMISE_IN_BOUNDS)


_G = 2048
_F = 128
_V = 100000
_CH = 80
_NCHK = _V // _CH
_NW = 32
_RPTM = 40


def _u_body(x_ref, g_ref, w1_ref, w2_ref, bl_ref, u_ref, c_ref):
    b = pl.program_id(0)

    @pl.when(b == 0)
    def _():
        gg = jnp.maximum(g_ref[...], 0.0)
        c_ref[...] = lax.dot_general(
            w1_ref[...], gg, (((0,), (1,)), ((), ())),
            preferred_element_type=jnp.float32) + bl_ref[...]

    u_ref[...] = jnp.dot(x_ref[...], w2_ref[...],
                         preferred_element_type=jnp.float32)


def _sc_body(x_hbm, u_hbm, seg_hbm, c_hbm, S_out, d_out,
             seg2d, u_fl, seg_fl, cseg_v, e_v, xb, zbuf, zd_v, S_sh, d_sh,
             sem_g, sem_in, sem_s, sem_d):
    cid = lax.axis_index("c")
    sid = lax.axis_index("s")
    wid = sid * 2 + cid

    def _zrow(i, carry):
        for j in range(_F // 16):
            zbuf[i, pl.ds(j * 16, 16)] = jnp.zeros((16,), jnp.float32)
        return carry
    lax.fori_loop(0, _F, _zrow, 0)

    def _zd(i, carry):
        zd_v[pl.ds(i * 16, 16)] = jnp.zeros((16,), jnp.float32)
        return carry
    lax.fori_loop(0, _G // 16, _zd, 0)

    stripe = pl.multiple_of(sid * _F, _F)
    pltpu.sync_copy(zbuf, S_sh.at[pl.ds(stripe, _F)])

    @pl.when(sid == 0)
    def _():
        pltpu.sync_copy(zd_v, d_sh)

    plsc.subcore_barrier()

    ncht = jnp.where(wid < 2, _RPTM, _RPTM - 1)
    cstart = wid * (_RPTM - 1) + jnp.minimum(wid, 2)
    nbase = pl.multiple_of(cstart * _CH, 16)
    npt = _RPTM * _CH
    pltpu.sync_copy(u_hbm.at[pl.ds(nbase, npt)], u_fl)
    pltpu.sync_copy(seg_hbm.at[pl.ds(nbase, npt)], seg_fl)

    def _rp(r, carry):
        for j in range(_CH // 16):
            seg2d[r, pl.ds(j * 16, 16)] = (
                seg_fl[pl.ds(r * _CH + j * 16, 16)])
        return carry
    lax.fori_loop(0, ncht, _rp, 0)

    descs = [
        pltpu.async_copy(c_hbm.at[seg2d.at[r]], cseg_v.at[r], sem_g)
        for r in range(_RPTM - 1)
    ]

    @pl.when(ncht == _RPTM)
    def _():
        pltpu.async_copy(c_hbm.at[seg2d.at[_RPTM - 1]],
                         cseg_v.at[_RPTM - 1], sem_g)

    for dsc in descs:
        dsc.wait()

    @pl.when(ncht == _RPTM)
    def _():
        pltpu.make_async_copy(c_hbm.at[seg2d.at[_RPTM - 1]],
                              cseg_v.at[_RPTM - 1], sem_g).wait()

    def _ebody(r, carry):
        for j in range(_CH // 16):
            fsl = pl.ds(r * _CH + j * 16, 16)
            z = u_fl[fsl] + cseg_v[r, pl.ds(j * 16, 16)]
            z = jnp.maximum(z, 0.01 * z)
            z = jnp.minimum(z, 80.0)
            e_v[fsl] = jnp.exp(z)
        return carry
    lax.fori_loop(0, ncht, _ebody, 0)

    def _xsl(r):
        return x_hbm.at[pl.ds(pl.multiple_of(nbase + r * _CH, 16),
                              _CH), :]

    pltpu.async_copy(_xsl(0), xb.at[0], sem_in)

    def _chunk(r, carry):
        cb = lax.rem(r, 3)

        @pl.when(r >= 2)
        def _():
            pltpu.make_async_copy(_xsl(0), xb.at[0], sem_s).wait()

        @pl.when(r + 1 < ncht)
        def _():
            pltpu.async_copy(_xsl(r + 1), xb.at[lax.rem(r + 1, 3)],
                             sem_in)

        pltpu.make_async_copy(_xsl(0), xb.at[0], sem_in).wait()

        xrow = xb.at[cb]
        rbase = r * _CH
        for g in range(_CH // 16):
            ev16 = e_v[pl.ds(rbase + g * 16, 16)]
            for k in range(16):
                i = g * 16 + k
                ev = _bcast_lane(ev16, k)
                for j in range(_F // 16):
                    sl = pl.ds(j * 16, 16)
                    xrow[i, sl] = xrow[i, sl] * ev
        pltpu.async_copy(xb.at[cb], S_sh.at[seg2d.at[r]], sem_s,
                         add=True)
        pltpu.async_copy(e_v.at[pl.ds(rbase, _CH)],
                         d_sh.at[seg2d.at[r]], sem_d, add=True)

        @pl.when(r >= 10)
        def _():
            pltpu.make_async_copy(u_hbm.at[pl.ds(nbase, _CH)],
                                  e_v.at[pl.ds(0, _CH)], sem_d).wait()
        return carry
    lax.fori_loop(0, ncht, _chunk, 0)

    for _ in range(2):
        pltpu.make_async_copy(_xsl(0), xb.at[0], sem_s).wait()
    for _ in range(10):
        pltpu.make_async_copy(u_hbm.at[pl.ds(nbase, _CH)],
                              e_v.at[pl.ds(0, _CH)], sem_d).wait()

    plsc.subcore_barrier()

    pltpu.sync_copy(S_sh.at[pl.ds(stripe, _F)],
                    S_out.at[cid, pl.ds(stripe, _F)])

    @pl.when(sid == 0)
    def _():
        dslot = pl.multiple_of(cid * _G, _G)
        pltpu.sync_copy(d_sh, d_out.at[pl.ds(dslot, _G)])


def _gru_body(S_ref, d_ref, g_ref, Wp_ref, bp_ref, Wih_ref, Whh_ref,
              bih_ref, bhh_ref, out_ref):
    S = S_ref[0] + S_ref[1]
    d = d_ref[0] + d_ref[1]
    mask = d > 0.0
    inv = jnp.where(mask, 1.0 / jnp.where(mask, d, 1.0), 0.0)
    M = S * inv
    g = g_ref[...]
    grepr = lax.dot_general(M, Wp_ref[...], (((1,), (1,)), ((), ())),
                            preferred_element_type=jnp.float32)
    grepr = grepr + jnp.where(mask, bp_ref[...], 0.0)
    ctx = jnp.where(grepr > 0.0,
                    grepr,
                    jnp.exp(jnp.minimum(grepr, 0.0)) - 1.0)
    gi = lax.dot_general(ctx, Wih_ref[...], (((1,), (1,)), ((), ())),
                         preferred_element_type=jnp.float32) + bih_ref[...]
    gh = lax.dot_general(g, Whh_ref[...], (((1,), (1,)), ((), ())),
                         preferred_element_type=jnp.float32) + bhh_ref[...]
    F = g.shape[1]
    i_r, i_z, i_n = gi[:, :F], gi[:, F:2 * F], gi[:, 2 * F:]
    h_r, h_z, h_n = gh[:, :F], gh[:, F:2 * F], gh[:, 2 * F:]
    r = 1.0 / (1.0 + jnp.exp(-(i_r + h_r)))
    zg = 1.0 / (1.0 + jnp.exp(-(i_z + h_z)))
    n = jnp.tanh(i_n + r * h_n)
    out_ref[...] = (1.0 - zg) * n + zg * g


def kernel(node_feats, g_feats, segment_ids, W_logits, b_logits,
           W_proj, b_proj, W_ih, W_hh, b_ih, b_hh):
    V, F = node_feats.shape
    G = g_feats.shape[0]
    B = 8192
    nb = pl.cdiv(V, B)
    Vp = nb * B
    w1 = W_logits[:, :F].reshape(F, 1)
    w2 = W_logits[:, F:].reshape(F, 1)
    bl = b_logits.reshape(1, 1)

    u, c = pl.pallas_call(
        _u_body,
        grid=(nb,),
        in_specs=[
            pl.BlockSpec((B, F), lambda b: (b, 0)),
            pl.BlockSpec((G, F), lambda b: (0, 0)),
            pl.BlockSpec((F, 1), lambda b: (0, 0)),
            pl.BlockSpec((F, 1), lambda b: (0, 0)),
            pl.BlockSpec((1, 1), lambda b: (0, 0)),
        ],
        out_specs=[
            pl.BlockSpec((B, 1), lambda b: (b, 0)),
            pl.BlockSpec((1, G), lambda b: (0, 0)),
        ],
        out_shape=[
            jax.ShapeDtypeStruct((Vp, 1), jnp.float32),
            jax.ShapeDtypeStruct((1, G), jnp.float32),
        ],
    )(node_feats, g_feats, w1, w2, bl)

    u1d = u.reshape(Vp)
    cflat = c.reshape(G)
    segp = jnp.zeros((Vp,), segment_ids.dtype).at[:V].set(segment_ids)

    mesh = plsc.VectorSubcoreMesh(core_axis_name="c", subcore_axis_name="s")
    Spart, dflat = pl.kernel(
        _sc_body,
        out_type=[
            jax.ShapeDtypeStruct((2, G, F), jnp.float32),
            jax.ShapeDtypeStruct((2 * G,), jnp.float32),
        ],
        mesh=mesh,
        scratch_types=[
            pltpu.VMEM((_RPTM, _CH), jnp.int32),
            pltpu.VMEM((_RPTM * _CH,), jnp.float32),
            pltpu.VMEM((_RPTM * _CH,), jnp.int32),
            pltpu.VMEM((_RPTM, _CH), jnp.float32),
            pltpu.VMEM((_RPTM * _CH,), jnp.float32),
            pltpu.VMEM((3, _CH, F), jnp.float32),
            pltpu.VMEM((F, F), jnp.float32),
            pltpu.VMEM((G,), jnp.float32),
            pltpu.VMEM_SHARED((G, F), jnp.float32),
            pltpu.VMEM_SHARED((G,), jnp.float32),
            pltpu.SemaphoreType.DMA,
            pltpu.SemaphoreType.DMA,
            pltpu.SemaphoreType.DMA,
            pltpu.SemaphoreType.DMA,
        ],
    )(node_feats, u1d, segp, cflat)

    Bg = 512
    ng = G // Bg
    bp = b_proj.reshape(1, F)
    bih = b_ih.reshape(1, 3 * F)
    bhh = b_hh.reshape(1, 3 * F)
    d3 = dflat.reshape(2, G, 1)

    out = pl.pallas_call(
        _gru_body,
        grid=(ng,),
        in_specs=[
            pl.BlockSpec((2, Bg, F), lambda b: (0, b, 0)),
            pl.BlockSpec((2, Bg, 1), lambda b: (0, b, 0)),
            pl.BlockSpec((Bg, F), lambda b: (b, 0)),
            pl.BlockSpec((F, F), lambda b: (0, 0)),
            pl.BlockSpec((1, F), lambda b: (0, 0)),
            pl.BlockSpec((3 * F, F), lambda b: (0, 0)),
            pl.BlockSpec((3 * F, F), lambda b: (0, 0)),
            pl.BlockSpec((1, 3 * F), lambda b: (0, 0)),
            pl.BlockSpec((1, 3 * F), lambda b: (0, 0)),
        ],
        out_specs=pl.BlockSpec((Bg, F), lambda b: (b, 0)),
        out_shape=jax.ShapeDtypeStruct((G, F), jnp.float32),
    )(Spart, d3, g_feats, W_proj, bp, W_ih, W_hh, bih, bhh)
    return out


# speedup vs baseline: 1.8744x; 1.0119x over previous
import functools

import jax
import jax.numpy as jnp
from jax import lax
from jax.experimental import pallas as pl
from jax.experimental.pallas import tpu as pltpu
from jax.experimental.pallas import tpu_sc as plsc

_BCAST_DNUMS = lax.GatherDimensionNumbers(
    offset_dims=(), collapsed_slice_dims=(0,), start_index_map=(0,))


def _bcast_lane(vec16, k):
    idx = jnp.full((16, 1), k, jnp.int32)
    return lax.gather(vec16, idx, _BCAST_DNUMS, (1,),
                      mode=lax.GatherScatterMode.PROMISE_IN_BOUNDS)


_G = 2048
_F = 128
_V = 100000
_CH = 80
_NCHK = _V // _CH
_NW = 32
_RPTM = 40


def _u_body(x_ref, g_ref, w1_ref, w2_ref, bl_ref, u_ref, c_ref):
    b = pl.program_id(0)

    @pl.when(b == 0)
    def _():
        gg = jnp.maximum(g_ref[...], 0.0)
        c_ref[...] = lax.dot_general(
            w1_ref[...], gg, (((0,), (1,)), ((), ())),
            preferred_element_type=jnp.float32) + bl_ref[...]

    u_ref[...] = jnp.dot(x_ref[...], w2_ref[...],
                         preferred_element_type=jnp.float32)


def _sc_body(x_hbm, u_hbm, seg_hbm, c_hbm, S_out, d_out,
             seg2d, u_fl, seg_fl, cseg_v, e_v, xb, zbuf, zd_v, S_sh, d_sh,
             sem_g, sem_in, sem_s, sem_d):
    cid = lax.axis_index("c")
    sid = lax.axis_index("s")
    wid = sid * 2 + cid

    def _zrow(i, carry):
        for j in range(_F // 16):
            zbuf[i, pl.ds(j * 16, 16)] = jnp.zeros((16,), jnp.float32)
        return carry
    lax.fori_loop(0, _F, _zrow, 0)

    def _zd(i, carry):
        zd_v[pl.ds(i * 16, 16)] = jnp.zeros((16,), jnp.float32)
        return carry
    lax.fori_loop(0, _G // 16, _zd, 0)

    stripe = pl.multiple_of(sid * _F, _F)
    pltpu.sync_copy(zbuf, S_sh.at[pl.ds(stripe, _F)])

    @pl.when(sid == 0)
    def _():
        pltpu.sync_copy(zd_v, d_sh)

    plsc.subcore_barrier()

    ncht = jnp.where(wid < 2, _RPTM, _RPTM - 1)
    cstart = wid * (_RPTM - 1) + jnp.minimum(wid, 2)
    nbase = pl.multiple_of(cstart * _CH, 16)
    npt = _RPTM * _CH
    pltpu.sync_copy(u_hbm.at[pl.ds(nbase, npt)], u_fl)
    pltpu.sync_copy(seg_hbm.at[pl.ds(nbase, npt)], seg_fl)

    def _rp(r, carry):
        for j in range(_CH // 16):
            seg2d[r, pl.ds(j * 16, 16)] = (
                seg_fl[pl.ds(r * _CH + j * 16, 16)])
        return carry
    lax.fori_loop(0, ncht, _rp, 0)

    descs = [
        pltpu.async_copy(c_hbm.at[seg2d.at[r]], cseg_v.at[r], sem_g)
        for r in range(_RPTM - 1)
    ]

    @pl.when(ncht == _RPTM)
    def _():
        pltpu.async_copy(c_hbm.at[seg2d.at[_RPTM - 1]],
                         cseg_v.at[_RPTM - 1], sem_g)

    for dsc in descs:
        dsc.wait()

    @pl.when(ncht == _RPTM)
    def _():
        pltpu.make_async_copy(c_hbm.at[seg2d.at[_RPTM - 1]],
                              cseg_v.at[_RPTM - 1], sem_g).wait()

    def _ebody(r, carry):
        for j in range(_CH // 16):
            fsl = pl.ds(r * _CH + j * 16, 16)
            z = u_fl[fsl] + cseg_v[r, pl.ds(j * 16, 16)]
            z = jnp.maximum(z, 0.01 * z)
            z = jnp.minimum(z, 80.0)
            e_v[fsl] = jnp.exp(z)
        return carry
    lax.fori_loop(0, ncht, _ebody, 0)

    def _xsl(r):
        return x_hbm.at[pl.ds(pl.multiple_of(nbase + r * _CH, 16),
                              _CH), :]

    pltpu.async_copy(_xsl(0), xb.at[0], sem_in)

    def _chunk(r, carry):
        cb = lax.rem(r, 3)

        @pl.when(r >= 2)
        def _():
            pltpu.make_async_copy(_xsl(0), xb.at[0], sem_s).wait()

        @pl.when(r + 1 < ncht)
        def _():
            pltpu.async_copy(_xsl(r + 1), xb.at[lax.rem(r + 1, 3)],
                             sem_in)

        pltpu.make_async_copy(_xsl(0), xb.at[0], sem_in).wait()

        xrow = xb.at[cb]
        rbase = r * _CH
        for g in range(_CH // 16):
            ev16 = e_v[pl.ds(rbase + g * 16, 16)]
            for k in range(16):
                i = g * 16 + k
                ev = _bcast_lane(ev16, k)
                for j in range(_F // 16):
                    sl = pl.ds(j * 16, 16)
                    xrow[i, sl] = xrow[i, sl] * ev
        pltpu.async_copy(xb.at[cb], S_sh.at[seg2d.at[r]], sem_s,
                         add=True)
        pltpu.async_copy(e_v.at[pl.ds(rbase, _CH)],
                         d_sh.at[seg2d.at[r]], sem_d, add=True)

        @pl.when(r >= 10)
        def _():
            pltpu.make_async_copy(u_hbm.at[pl.ds(nbase, _CH)],
                                  e_v.at[pl.ds(0, _CH)], sem_d).wait()
        return carry
    lax.fori_loop(0, ncht, _chunk, 0)

    for _ in range(2):
        pltpu.make_async_copy(_xsl(0), xb.at[0], sem_s).wait()
    for _ in range(10):
        pltpu.make_async_copy(u_hbm.at[pl.ds(nbase, _CH)],
                              e_v.at[pl.ds(0, _CH)], sem_d).wait()

    plsc.subcore_barrier()

    pltpu.sync_copy(S_sh.at[pl.ds(stripe, _F)],
                    S_out.at[cid, pl.ds(stripe, _F)])

    @pl.when(sid == 0)
    def _():
        dslot = pl.multiple_of(cid * _G, _G)
        pltpu.sync_copy(d_sh, d_out.at[pl.ds(dslot, _G)])


def _gru_body(S_ref, d_ref, g_ref, Wp_ref, bp_ref, Wih_ref, Whh_ref,
              bih_ref, bhh_ref, out_ref):
    S = S_ref[0] + S_ref[1]
    d = d_ref[0] + d_ref[1]
    mask = d > 0.0
    inv = jnp.where(mask, 1.0 / jnp.where(mask, d, 1.0), 0.0)
    M = S * inv
    g = g_ref[...]
    grepr = lax.dot_general(M, Wp_ref[...], (((1,), (1,)), ((), ())),
                            preferred_element_type=jnp.float32)
    grepr = grepr + jnp.where(mask, bp_ref[...], 0.0)
    ctx = jnp.where(grepr > 0.0,
                    grepr,
                    jnp.exp(jnp.minimum(grepr, 0.0)) - 1.0)
    gi = lax.dot_general(ctx, Wih_ref[...], (((1,), (1,)), ((), ())),
                         preferred_element_type=jnp.float32) + bih_ref[...]
    gh = lax.dot_general(g, Whh_ref[...], (((1,), (1,)), ((), ())),
                         preferred_element_type=jnp.float32) + bhh_ref[...]
    F = g.shape[1]
    i_r, i_z, i_n = gi[:, :F], gi[:, F:2 * F], gi[:, 2 * F:]
    h_r, h_z, h_n = gh[:, :F], gh[:, F:2 * F], gh[:, 2 * F:]
    r = 1.0 / (1.0 + jnp.exp(-(i_r + h_r)))
    zg = 1.0 / (1.0 + jnp.exp(-(i_z + h_z)))
    n = jnp.tanh(i_n + r * h_n)
    out_ref[...] = (1.0 - zg) * n + zg * g


def kernel(node_feats, g_feats, segment_ids, W_logits, b_logits,
           W_proj, b_proj, W_ih, W_hh, b_ih, b_hh):
    V, F = node_feats.shape
    G = g_feats.shape[0]
    B = 12800
    nb = pl.cdiv(V, B)
    Vp = nb * B
    w1 = W_logits[:, :F].reshape(F, 1)
    w2 = W_logits[:, F:].reshape(F, 1)
    bl = b_logits.reshape(1, 1)

    u, c = pl.pallas_call(
        _u_body,
        grid=(nb,),
        in_specs=[
            pl.BlockSpec((B, F), lambda b: (b, 0)),
            pl.BlockSpec((G, F), lambda b: (0, 0)),
            pl.BlockSpec((F, 1), lambda b: (0, 0)),
            pl.BlockSpec((F, 1), lambda b: (0, 0)),
            pl.BlockSpec((1, 1), lambda b: (0, 0)),
        ],
        out_specs=[
            pl.BlockSpec((B, 1), lambda b: (b, 0)),
            pl.BlockSpec((1, G), lambda b: (0, 0)),
        ],
        out_shape=[
            jax.ShapeDtypeStruct((Vp, 1), jnp.float32),
            jax.ShapeDtypeStruct((1, G), jnp.float32),
        ],
    )(node_feats, g_feats, w1, w2, bl)

    u1d = u.reshape(Vp)
    cflat = c.reshape(G)
    segp = jnp.zeros((Vp,), segment_ids.dtype).at[:V].set(segment_ids)

    mesh = plsc.VectorSubcoreMesh(core_axis_name="c", subcore_axis_name="s")
    Spart, dflat = pl.kernel(
        _sc_body,
        out_type=[
            jax.ShapeDtypeStruct((2, G, F), jnp.float32),
            jax.ShapeDtypeStruct((2 * G,), jnp.float32),
        ],
        mesh=mesh,
        scratch_types=[
            pltpu.VMEM((_RPTM, _CH), jnp.int32),
            pltpu.VMEM((_RPTM * _CH,), jnp.float32),
            pltpu.VMEM((_RPTM * _CH,), jnp.int32),
            pltpu.VMEM((_RPTM, _CH), jnp.float32),
            pltpu.VMEM((_RPTM * _CH,), jnp.float32),
            pltpu.VMEM((3, _CH, F), jnp.float32),
            pltpu.VMEM((F, F), jnp.float32),
            pltpu.VMEM((G,), jnp.float32),
            pltpu.VMEM_SHARED((G, F), jnp.float32),
            pltpu.VMEM_SHARED((G,), jnp.float32),
            pltpu.SemaphoreType.DMA,
            pltpu.SemaphoreType.DMA,
            pltpu.SemaphoreType.DMA,
            pltpu.SemaphoreType.DMA,
        ],
    )(node_feats, u1d, segp, cflat)

    Bg = 512
    ng = G // Bg
    bp = b_proj.reshape(1, F)
    bih = b_ih.reshape(1, 3 * F)
    bhh = b_hh.reshape(1, 3 * F)
    d3 = dflat.reshape(2, G, 1)

    out = pl.pallas_call(
        _gru_body,
        grid=(ng,),
        in_specs=[
            pl.BlockSpec((2, Bg, F), lambda b: (0, b, 0)),
            pl.BlockSpec((2, Bg, 1), lambda b: (0, b, 0)),
            pl.BlockSpec((Bg, F), lambda b: (b, 0)),
            pl.BlockSpec((F, F), lambda b: (0, 0)),
            pl.BlockSpec((1, F), lambda b: (0, 0)),
            pl.BlockSpec((3 * F, F), lambda b: (0, 0)),
            pl.BlockSpec((3 * F, F), lambda b: (0, 0)),
            pl.BlockSpec((1, 3 * F), lambda b: (0, 0)),
            pl.BlockSpec((1, 3 * F), lambda b: (0, 0)),
        ],
        out_specs=pl.BlockSpec((Bg, F), lambda b: (b, 0)),
        out_shape=jax.ShapeDtypeStruct((G, F), jnp.float32),
    )(Spart, d3, g_feats, W_proj, bp, W_ih, W_hh, bih, bhh)
    return out
